# baseline (speedup 1.0000x reference)
import functools

import jax
import jax.numpy as jnp
from jax import lax
from jax.experimental import pallas as pl
from jax.experimental.pallas import tpu as pltpu
from jax.experimental.pallas import tpu_sc as plsc

N_NODES = 10000
N_EDGES = 160000
DIM = 128
HEADS = 4
CH = 128

NPAD = 10240
EPAD = 171008
NCORES = 2
NSUB = 16
EDGES_PER_TILE = EPAD // NSUB
CHUNK = 64
NCHTOT = EPAD // CHUNK
ROWS_PER_TILE = NPAD // NSUB
DEN_ROWS = (ROWS_PER_TILE + 1 + 7) // 8
VPH = CH // 16

XBLK = 2048
NXBLK = NPAD // XBLK
EABLK = N_EDGES // NXBLK


def _k1_body(x_ref, wl_ref, bl_ref, wr_ref, br_ref, eattr_ref,
             xl_ref, xr_ref, ea_ref):
    h = pl.program_id(0)
    xb = x_ref[...]
    xl_ref[0] = jnp.dot(xb, wl_ref[...], preferred_element_type=jnp.float32) \
        + bl_ref[pl.ds(h, 1), :]
    xr_ref[0] = jnp.dot(xb, wr_ref[...], preferred_element_type=jnp.float32) \
        + br_ref[pl.ds(h, 1), :]
    m = jnp.mean(eattr_ref[0], axis=-1)
    ea_ref[0] = m.reshape(EABLK // 128, 128)


def _project(xpad, W_l, b_l, W_r, b_r, eattr3):
    grid = (HEADS, NXBLK)
    return pl.pallas_call(
        _k1_body,
        grid=grid,
        in_specs=[
            pl.BlockSpec((XBLK, DIM), lambda h, i: (i, 0)),
            pl.BlockSpec((DIM, CH), lambda h, i: (0, h)),
            pl.BlockSpec((HEADS, CH), lambda h, i: (0, 0)),
            pl.BlockSpec((DIM, CH), lambda h, i: (0, h)),
            pl.BlockSpec((HEADS, CH), lambda h, i: (0, 0)),
            pl.BlockSpec((1, EABLK, 4), lambda h, i: (i, 0, 0)),
        ],
        out_specs=[
            pl.BlockSpec((1, XBLK, CH), lambda h, i: (h, i, 0)),
            pl.BlockSpec((1, XBLK, CH), lambda h, i: (h, i, 0)),
            pl.BlockSpec((1, EABLK // 128, 128), lambda h, i: (i, 0, 0)),
        ],
        out_shape=[
            jax.ShapeDtypeStruct((HEADS, NPAD, CH), jnp.float32),
            jax.ShapeDtypeStruct((HEADS, NPAD, CH), jnp.float32),
            jax.ShapeDtypeStruct((NXBLK, EABLK // 128, 128), jnp.float32),
        ],
    )(xpad, W_l, b_l.reshape(HEADS, CH), W_r, b_r.reshape(HEADS, CH), eattr3)


def _sc_edge_kernel(xl_f, xr_f, srcoff_h, dstoff_h, dst_h, ea_h, par_f,
                    out_f, den_f,
                    a_v, b_v, di_v, sif_v, dif_v, ea_v, par_v, t_v,
                    acc_v, den_v):
    cid = lax.axis_index("c")
    sid = lax.axis_index("s")
    tflat = cid * NSUB + sid

    for k in range(2):
        pid = tflat * 2 + k
        h = pid // 16
        r = pid % 16
        rbase = r * ROWS_PER_TILE

        pltpu.sync_copy(par_f.at[pl.ds(h * 8, 8)], par_v)

        zz = jnp.zeros((16,), jnp.float32)

        def zero_body(row, carry):
            for v in range(VPH):
                acc_v[row, pl.ds(v * 16, 16)] = zz
            return carry

        lax.fori_loop(0, ROWS_PER_TILE + 1, zero_body, 0)

        def zero_den(row, carry):
            for v in range(VPH):
                den_v[row, pl.ds(v * 16, 16)] = zz
            return carry

        lax.fori_loop(0, DEN_ROWS, zero_den, 0)

        wv = [par_v[0, pl.ds(v * 16, 16)] for v in range(VPH)]
        av = [par_v[1, pl.ds(v * 16, 16)] for v in range(VPH)]

        def chunk_body(c, carry):
            off = c * CHUNK
            hoffe = h * EPAD + off
            pltpu.sync_copy(srcoff_h.at[pl.ds(hoffe, CHUNK)], sif_v)
            pltpu.sync_copy(dstoff_h.at[pl.ds(hoffe, CHUNK)], dif_v)
            pltpu.sync_copy(dst_h.at[pl.ds(off, CHUNK)], di_v)
            pltpu.sync_copy(ea_h.at[pl.ds(off, CHUNK)], ea_v)
            pltpu.sync_copy(xl_f.at[sif_v], a_v)
            pltpu.sync_copy(xr_f.at[dif_v], b_v)

            def group_body(g, carry2):
                eas = ea_v[pl.ds(g * 16, 16)]
                dloc = di_v[pl.ds(g * 16, 16)] - rbase
                own = (dloc >= 0) & (dloc < ROWS_PER_TILE)
                lrow16 = jnp.where(own, dloc, ROWS_PER_TILE)
                for j in range(16):
                    e = g * 16 + j
                    eae = eas[j]
                    t = jnp.zeros((16,), jnp.float32)
                    for v in range(VPH):
                        sl = pl.ds(v * 16, 16)
                        m = a_v[e, sl] + b_v[e, sl] + eae * wv[v]
                        m = jnp.where(m >= 0.0, m, 0.2 * m)
                        t = t + m * av[v]
                    t_v[j] = t
                iot = lax.iota(jnp.int32, 16)
                acc16 = jnp.zeros((16,), jnp.float32)
                for v in range(16):
                    col = plsc.load_gather(
                        t_v, [iot, jnp.full((16,), v, jnp.int32)])
                    acc16 = acc16 + col
                aa16 = jnp.exp(acc16)
                for j in range(16):
                    e = g * 16 + j
                    aj = aa16[j]
                    lrow = lrow16[j]
                    for v in range(VPH):
                        sl = pl.ds(v * 16, 16)
                        acc_v[lrow, sl] = acc_v[lrow, sl] + a_v[e, sl] * aj
                    drow = lax.shift_right_logical(lrow, 3)
                    dsl = pl.ds(lax.shift_left(lrow & 7, 4), 16)
                    den_v[drow, dsl] = den_v[drow, dsl] \
                        + jnp.broadcast_to(aj, (16,))
                return carry2

            lax.fori_loop(0, CHUNK // 16, group_body, 0)
            return carry

        lax.fori_loop(0, NCHTOT, chunk_body, 0)
        obase = h * NPAD + r * ROWS_PER_TILE
        obase8 = h * (NPAD // 8) + r * (ROWS_PER_TILE // 8)
        pltpu.sync_copy(acc_v.at[pl.ds(0, ROWS_PER_TILE)],
                        out_f.at[pl.ds(obase, ROWS_PER_TILE)])
        pltpu.sync_copy(den_v.at[pl.ds(0, ROWS_PER_TILE // 8)],
                        den_f.at[pl.ds(obase8, ROWS_PER_TILE // 8)])


_sc_edge = functools.partial(
    pl.kernel,
    out_type=(
        jax.ShapeDtypeStruct((HEADS * NPAD, CH), jnp.float32),
        jax.ShapeDtypeStruct((HEADS * NPAD // 8, 128), jnp.float32),
    ),
    mesh=plsc.VectorSubcoreMesh(
        core_axis_name="c", subcore_axis_name="s",
        num_cores=NCORES, num_subcores=NSUB),
    scratch_types=[
        pltpu.VMEM((CHUNK, CH), jnp.float32),
        pltpu.VMEM((CHUNK, CH), jnp.float32),
        pltpu.VMEM((CHUNK,), jnp.int32),
        pltpu.VMEM((CHUNK,), jnp.int32),
        pltpu.VMEM((CHUNK,), jnp.int32),
        pltpu.VMEM((CHUNK,), jnp.float32),
        pltpu.VMEM((8, CH), jnp.float32),
        pltpu.VMEM((16, 16), jnp.float32),
        pltpu.VMEM((ROWS_PER_TILE + 1, CH), jnp.float32),
        pltpu.VMEM((DEN_ROWS, 128), jnp.float32),
    ],
    compiler_params=pltpu.CompilerParams(needs_layout_passes=False),
)(_sc_edge_kernel)


def _k2_body(out_ref, den_ref, x_ref, bias_ref, wp_ref, bp_ref, g_ref, be_ref,
             o_ref):
    acc = jnp.zeros((XBLK, DIM), jnp.float32)
    for h in range(HEADS):
        hv = out_ref[h] / (den_ref[h][:, 0:1] + 1e-16) \
            + bias_ref[h].reshape(1, CH)
        acc = acc + jnp.dot(hv, wp_ref[h], preferred_element_type=jnp.float32)
    hres = acc + bp_ref[...] + x_ref[...]
    hres = jnp.where(hres > 0.0, hres, jnp.exp(hres) - 1.0)
    mu = jnp.mean(hres, axis=-1, keepdims=True)
    var = jnp.mean(hres * hres, axis=-1, keepdims=True) - mu * mu
    o_ref[...] = (hres - mu) * lax.rsqrt(var + 1e-5) * g_ref[...] + be_ref[...]


def _finalize(out4, den4, xpad, bias_out, W_p, b_p, gamma, beta):
    return pl.pallas_call(
        _k2_body,
        grid=(NXBLK,),
        in_specs=[
            pl.BlockSpec((HEADS, XBLK, CH), lambda i: (0, i, 0)),
            pl.BlockSpec((HEADS, XBLK, 16), lambda i: (0, i, 0)),
            pl.BlockSpec((XBLK, DIM), lambda i: (i, 0)),
            pl.BlockSpec((HEADS, CH), lambda i: (0, 0)),
            pl.BlockSpec((HEADS, CH, DIM), lambda i: (0, 0, 0)),
            pl.BlockSpec((1, DIM), lambda i: (0, 0)),
            pl.BlockSpec((1, DIM), lambda i: (0, 0)),
            pl.BlockSpec((1, DIM), lambda i: (0, 0)),
        ],
        out_specs=pl.BlockSpec((XBLK, DIM), lambda i: (i, 0)),
        out_shape=jax.ShapeDtypeStruct((NPAD, DIM), jnp.float32),
    )(out4, den4, xpad, bias_out.reshape(HEADS, CH),
      W_p.reshape(HEADS, CH, DIM), b_p.reshape(1, DIM),
      gamma.reshape(1, DIM), beta.reshape(1, DIM))


def kernel(x, edge_index, edge_attr, W_l, b_l, W_r, b_r, W_e, att,
           bias_out, W_p, b_p, gamma, beta):
    f32 = jnp.float32
    xpad = jnp.zeros((NPAD, DIM), f32).at[:N_NODES].set(x)
    eattr3 = edge_attr.reshape(NXBLK, EABLK, 4)

    xl4, xr4, ea3 = _project(xpad, W_l, b_l, W_r, b_r, eattr3)
    ea_e = ea3.reshape(N_EDGES)
    ea_mean = jnp.mean(ea_e)

    loop_idx = jnp.arange(N_NODES, dtype=jnp.int32)
    padn = EPAD - N_EDGES - N_NODES
    src_full = jnp.concatenate([
        edge_index[0].astype(jnp.int32), loop_idx,
        jnp.full((padn,), N_NODES, jnp.int32)])
    dst_full = jnp.concatenate([
        edge_index[1].astype(jnp.int32), loop_idx,
        jnp.full((padn,), N_NODES, jnp.int32)])
    ea_full = jnp.concatenate([
        ea_e, jnp.full((N_NODES,), ea_mean, f32), jnp.zeros((padn,), f32)])

    par = jnp.zeros((HEADS, 8, CH), f32)
    par = par.at[:, 0].set(W_e.reshape(HEADS, CH))
    par = par.at[:, 1].set(att.astype(f32))
    par = par.reshape(HEADS * 8, CH)

    hoffs = (jnp.arange(HEADS, dtype=jnp.int32) * NPAD)[:, None]
    srcoff = (src_full[None, :] + hoffs).reshape(HEADS * EPAD)
    dstoff = (dst_full[None, :] + hoffs).reshape(HEADS * EPAD)

    out_f, den_f = _sc_edge(
        xl4.reshape(HEADS * NPAD, CH), xr4.reshape(HEADS * NPAD, CH),
        srcoff, dstoff, dst_full, ea_full, par)

    den_b = den_f.reshape(HEADS, NPAD // 8, 8, 16)[:, :, :, 0]
    den16 = jnp.broadcast_to(
        den_b.reshape(HEADS, NPAD, 1), (HEADS, NPAD, 16))
    res = _finalize(out_f.reshape(HEADS, NPAD, CH),
                    den16,
                    xpad, bias_out, W_p, b_p, gamma, beta)
    return res[:N_NODES]
